# Initial kernel scaffold; baseline (speedup 1.0000x reference)
import jax
import jax.numpy as jnp
from jax.experimental import pallas as pl


def kernel(groups_xy, W1, b1, g1, be1, W2, b2, g2, be2):
    raise NotImplementedError("write your pallas kernel here")


# speedup vs baseline: 28.4453x; 28.4453x over previous
import jax
import jax.numpy as jnp
from jax.experimental import pallas as pl

K = 4
B, M, NPTS = 8, 4096, 32
BM = B * M
G = 512


def _body(x_ref, y_ref, rcs_ref, vr_ref,
          w1_ref, b1_ref, g1_ref, be1_ref,
          w2_ref, b2_ref, g2_ref, be2_ref,
          out_ref):
    x = x_ref[...]
    y = y_ref[...]
    rcs = rcs_ref[...]
    vr = vr_ref[...]

    rng = jnp.hypot(x, y)
    az = jnp.arctan2(y, x)

    r_lo = jnp.min(rng, axis=0, keepdims=True)
    r_hi = jnp.max(rng, axis=0, keepdims=True)
    a_lo = jnp.min(az, axis=0, keepdims=True)
    a_hi = jnp.max(az, axis=0, keepdims=True)
    ur = (r_hi - r_lo) / K
    ua = (a_hi - a_lo) / K
    ur = jnp.where(ur == 0, 1.0, ur)
    ua = jnp.where(ua == 0, 1.0, ua)
    ridx = jnp.floor((rng - r_lo) / ur).astype(jnp.int32)
    aidx = jnp.floor((az - a_lo) / ua).astype(jnp.int32)
    ridx = jnp.clip(jnp.where(ridx == K, K - 1, ridx), 0, K - 1)
    aidx = jnp.clip(jnp.where(aidx == K, K - 1, aidx), 0, K - 1)
    flat = ridx * K + aidx

    cnt_rows, c1_rows, c2_rows = [], [], []
    for k in range(K * K):
        mask = flat == k
        cnt_rows.append(jnp.sum(mask.astype(jnp.float32), axis=0, keepdims=True))
        c1_rows.append(jnp.max(jnp.where(mask, rcs, 0.0), axis=0, keepdims=True))
        c2_rows.append(jnp.max(jnp.where(mask, vr, 0.0), axis=0, keepdims=True))
    counts = jnp.concatenate(cnt_rows, axis=0)
    ch1 = jnp.concatenate(c1_rows, axis=0)
    ch2 = jnp.concatenate(c2_rows, axis=0)

    h1 = [w1_ref[c, 0] * counts + w1_ref[c, 1] * ch1 + w1_ref[c, 2] * ch2
          + b1_ref[c, 0] for c in range(32)]
    hn = []
    for gi in range(8):
        grp = h1[4 * gi:4 * gi + 4]
        mean = jnp.sum(grp[0] + grp[1] + grp[2] + grp[3],
                       axis=0, keepdims=True) / 64.0
        d = [a - mean for a in grp]
        var = jnp.sum(d[0] * d[0] + d[1] * d[1] + d[2] * d[2] + d[3] * d[3],
                      axis=0, keepdims=True) / 64.0
        rstd = jax.lax.rsqrt(var + 1e-5)
        for j in range(4):
            c = 4 * gi + j
            hn.append(jnp.maximum(d[j] * rstd * g1_ref[c, 0] + be1_ref[c, 0],
                                  0.0))
    h = jnp.concatenate(hn, axis=0)

    o = jax.lax.dot_general(w2_ref[...], h, (((1,), (0,)), ((), ())),
                            preferred_element_type=jnp.float32)
    o = o + b2_ref[...]

    outs = []
    for gi in range(8):
        seg = o[8 * gi:8 * gi + 8]
        mean = jnp.mean(seg, axis=0, keepdims=True)
        dd = seg - mean
        var = jnp.mean(dd * dd, axis=0, keepdims=True)
        outs.append(dd * jax.lax.rsqrt(var + 1e-5))
    on = jnp.concatenate(outs, axis=0)
    out_ref[...] = jnp.maximum(on * g2_ref[...] + be2_ref[...], 0.0)


def _run(x, y, rcs, vr, w1, b1, g1, be1, w2f, b2, g2, be2, interpret=False):
    grid = BM // G
    whole = lambda s: pl.BlockSpec(s, lambda i: (0, 0))
    return pl.pallas_call(
        _body,
        grid=(grid,),
        in_specs=[
            pl.BlockSpec((NPTS, G), lambda i: (0, i)),
            pl.BlockSpec((NPTS, G), lambda i: (0, i)),
            pl.BlockSpec((NPTS, G), lambda i: (0, i)),
            pl.BlockSpec((NPTS, G), lambda i: (0, i)),
            whole((32, 3)), whole((32, 1)), whole((32, 1)), whole((32, 1)),
            whole((64, 512)), whole((64, 1)), whole((64, 1)), whole((64, 1)),
        ],
        out_specs=pl.BlockSpec((64, G), lambda i: (0, i)),
        out_shape=jax.ShapeDtypeStruct((64, BM), jnp.float32),
        interpret=interpret,
    )(x, y, rcs, vr, w1, b1, g1, be1, w2f, b2, g2, be2)


def kernel(groups_xy, W1, b1, g1, be1, W2, b2, g2, be2):
    g = groups_xy.reshape(BM, NPTS, 6)
    x = g[:, :, 0].T
    y = g[:, :, 1].T
    rcs = g[:, :, 3].T
    vr = g[:, :, 5].T
    w1 = W1.reshape(32, 3)
    w2f = W2.reshape(64, 512)
    col = lambda v: v.reshape(-1, 1)
    out = _run(x, y, rcs, vr, w1, col(b1), col(g1), col(be1),
               w2f, col(b2), col(g2), col(be2))
    return out.T.reshape(B, M, 64)
